# baseline (speedup 1.0000x reference)
import functools

import jax
import jax.numpy as jnp
from jax.experimental import pallas as pl
from jax.experimental.pallas import tpu as pltpu

_REL = [
    ("created_by", "Asset", "Creator"),
    ("licensed_to", "Asset", "Licensee"),
    ("similar_to", "Asset", "Asset"),
    ("flagged_with", "Asset", "Asset"),
    ("rev_created_by", "Creator", "Asset"),
    ("rev_licensed_to", "Licensee", "Asset"),
]


def _cdiv(a, b):
    return (a + b - 1) // b


def _mm(x, Wt, bias=None, acc=None, relu=False, block_m=512):
    M, K = x.shape
    N = Wt.shape[1]
    bm = min(block_m, M)
    grid = (_cdiv(M, bm),)

    has_bias = bias is not None
    has_acc = acc is not None

    def body(*refs):
        i = 0
        x_ref = refs[i]; i += 1
        w_ref = refs[i]; i += 1
        b_ref = None
        a_ref = None
        if has_bias:
            b_ref = refs[i]; i += 1
        if has_acc:
            a_ref = refs[i]; i += 1
        o_ref = refs[i]
        y = jnp.dot(x_ref[...], w_ref[...], preferred_element_type=jnp.float32)
        if has_bias:
            y = y + b_ref[...]
        if has_acc:
            y = y + a_ref[...]
        if relu:
            y = jnp.maximum(y, 0.0)
        o_ref[...] = y

    in_specs = [
        pl.BlockSpec((bm, K), lambda i: (i, 0)),
        pl.BlockSpec((K, N), lambda i: (0, 0)),
    ]
    ops = [x, Wt]
    if has_bias:
        in_specs.append(pl.BlockSpec((1, N), lambda i: (0, 0)))
        ops.append(bias.reshape(1, N))
    if has_acc:
        in_specs.append(pl.BlockSpec((bm, N), lambda i: (i, 0)))
        ops.append(acc)

    return pl.pallas_call(
        body,
        grid=grid,
        in_specs=in_specs,
        out_specs=pl.BlockSpec((bm, N), lambda i: (i, 0)),
        out_shape=jax.ShapeDtypeStruct((M, N), jnp.float32),
    )(*ops)


def _seg_mean(x_src, ei, n_dst):
    m = jnp.take(x_src, ei[0], axis=0)
    s = jax.ops.segment_sum(m, ei[1], num_segments=n_dst)
    c = jax.ops.segment_sum(jnp.ones((ei.shape[1],), x_src.dtype), ei[1],
                            num_segments=n_dst)
    return s / jnp.maximum(c, 1.0)[:, None]


def _heads(h2_A, h2_C, params):
    OUT = h2_A.shape[1]
    q = h2_A[0:1, :]
    W1, b1 = params["inf1"]
    W2, b2 = params["inf2"]
    A1, ab1 = params["att1"]
    A2, ab2 = params["att2"]

    def inf_body(q_ref, w1_ref, b1_ref, w2_ref, b2_ref, o_ref):
        t = jnp.dot(q_ref[...], w1_ref[...], preferred_element_type=jnp.float32)
        t = jnp.maximum(t + b1_ref[...], 0.0)
        o_ref[...] = jnp.dot(t, w2_ref[...],
                             preferred_element_type=jnp.float32) + b2_ref[...]

    inf = pl.pallas_call(
        inf_body,
        out_shape=jax.ShapeDtypeStruct((1, 1), jnp.float32),
    )(q, W1.T, b1.reshape(1, OUT), W2.T, b2.reshape(1, 1))

    A1q = A1[:, :OUT]
    A1c = A1[:, OUT:]
    NC = h2_C.shape[0]
    bm = 1024

    def att_body(c_ref, a1c_ref, qp_ref, a2_ref, ab2_ref, o_ref):
        t = jnp.dot(c_ref[...], a1c_ref[...], preferred_element_type=jnp.float32)
        t = jnp.maximum(t + qp_ref[...], 0.0)
        o_ref[...] = jnp.dot(t, a2_ref[...],
                             preferred_element_type=jnp.float32) + ab2_ref[...]

    qpart = _mm(q, A1q.T, bias=ab1)
    att = pl.pallas_call(
        att_body,
        grid=(_cdiv(NC, bm),),
        in_specs=[
            pl.BlockSpec((bm, OUT), lambda i: (i, 0)),
            pl.BlockSpec((OUT, OUT), lambda i: (0, 0)),
            pl.BlockSpec((1, OUT), lambda i: (0, 0)),
            pl.BlockSpec((OUT, 1), lambda i: (0, 0)),
            pl.BlockSpec((1, 1), lambda i: (0, 0)),
        ],
        out_specs=pl.BlockSpec((bm, 1), lambda i: (i, 0)),
        out_shape=jax.ShapeDtypeStruct((NC, 1), jnp.float32),
    )(h2_C, A1c.T, qpart, A2.T, ab2.reshape(1, 1))

    return inf.reshape(()), att[:, 0]


def kernel(x_Asset, x_Creator, x_Licensee, params, ei_created_by,
           ei_licensed_to, ei_similar_to, ei_flagged_with, ei_rev_created_by,
           ei_rev_licensed_to):
    eis = {
        "created_by": ei_created_by,
        "licensed_to": ei_licensed_to,
        "similar_to": ei_similar_to,
        "flagged_with": ei_flagged_with,
        "rev_created_by": ei_rev_created_by,
        "rev_licensed_to": ei_rev_licensed_to,
    }
    nn = {"Asset": x_Asset.shape[0], "Creator": x_Creator.shape[0],
          "Licensee": x_Licensee.shape[0]}
    p = params

    Wa, ba = p["asset_proj"]
    Wc, bc = p["creator_proj"]
    Wl_, bl_ = p["licensee_proj"]
    h = {
        "Asset": _mm(x_Asset, Wa.T, bias=ba),
        "Creator": _mm(x_Creator, Wc.T, bias=bc),
        "Licensee": _mm(x_Licensee, Wl_.T, bias=bl_),
    }

    for lp in (p["conv1"], p["conv2"]):
        Wr_sum, bl_sum, mean_acc = {}, {}, {}
        for name, st, dt in _REL:
            mean = _seg_mean(h[st], eis[name], nn[dt])
            if dt in mean_acc:
                mean_acc[dt] = _mm(mean, lp[name]["Wl"].T, acc=mean_acc[dt])
                Wr_sum[dt] = Wr_sum[dt] + lp[name]["Wr"]
                bl_sum[dt] = bl_sum[dt] + lp[name]["bl"]
            else:
                mean_acc[dt] = _mm(mean, lp[name]["Wl"].T)
                Wr_sum[dt] = lp[name]["Wr"]
                bl_sum[dt] = lp[name]["bl"]
        h = {
            dt: _mm(h[dt], Wr_sum[dt].T, bias=bl_sum[dt], acc=mean_acc[dt],
                    relu=True)
            for dt in mean_acc
        }

    inf, att = _heads(h["Asset"], h["Creator"], params)
    return (inf, att, h["Asset"], h["Creator"], h["Licensee"])


# speedup vs baseline: 1.1180x; 1.1180x over previous
import functools

import jax
import jax.numpy as jnp
from jax import lax
from jax.experimental import pallas as pl
from jax.experimental.pallas import tpu as pltpu
from jax.experimental.pallas import tpu_sc as plsc

_REL = [
    ("created_by", "Asset", "Creator"),
    ("licensed_to", "Asset", "Licensee"),
    ("similar_to", "Asset", "Asset"),
    ("flagged_with", "Asset", "Asset"),
    ("rev_created_by", "Creator", "Asset"),
    ("rev_licensed_to", "Licensee", "Asset"),
]


def _cdiv(a, b):
    return (a + b - 1) // b


def _mm(x, Wt, bias=None, acc=None, relu=False, block_m=512):
    M, K = x.shape
    N = Wt.shape[1]
    bm = min(block_m, M)
    grid = (_cdiv(M, bm),)

    has_bias = bias is not None
    has_acc = acc is not None

    def body(*refs):
        i = 0
        x_ref = refs[i]; i += 1
        w_ref = refs[i]; i += 1
        b_ref = None
        a_ref = None
        if has_bias:
            b_ref = refs[i]; i += 1
        if has_acc:
            a_ref = refs[i]; i += 1
        o_ref = refs[i]
        if K <= 4:
            y = None
            for k in range(K):
                t = x_ref[:, k:k + 1] * w_ref[k:k + 1, :]
                y = t if y is None else y + t
        else:
            y = None
            for k0 in range(0, K, 256):
                kw = min(256, K - k0)
                t = jnp.dot(x_ref[:, k0:k0 + kw], w_ref[k0:k0 + kw, :],
                            preferred_element_type=jnp.float32)
                y = t if y is None else y + t
        if has_bias:
            y = y + b_ref[...]
        if has_acc:
            y = y + a_ref[...]
        if relu:
            y = jnp.maximum(y, 0.0)
        o_ref[...] = y

    in_specs = [
        pl.BlockSpec((bm, K), lambda i: (i, 0)),
        pl.BlockSpec((K, N), lambda i: (0, 0)),
    ]
    ops = [x, Wt]
    if has_bias:
        in_specs.append(pl.BlockSpec((1, N), lambda i: (0, 0)))
        ops.append(bias.reshape(1, N))
    if has_acc:
        in_specs.append(pl.BlockSpec((bm, N), lambda i: (i, 0)))
        ops.append(acc)

    return pl.pallas_call(
        body,
        grid=grid,
        in_specs=in_specs,
        out_specs=pl.BlockSpec((bm, N), lambda i: (i, 0)),
        out_shape=jax.ShapeDtypeStruct((M, N), jnp.float32),
    )(*ops)


def _sage_mm(mean, WlT, bl, x, WrT, block_m=512):
    M, K1 = mean.shape
    N = WlT.shape[1]
    K2 = x.shape[1]
    bm = min(block_m, M)

    def body(m_ref, wl_ref, b_ref, x_ref, wr_ref, o_ref):
        t = None
        for k0 in range(0, K1, 256):
            kw = min(256, K1 - k0)
            y = jnp.dot(m_ref[:, k0:k0 + kw], wl_ref[k0:k0 + kw, :],
                        preferred_element_type=jnp.float32)
            t = y if t is None else t + y
        t = t + b_ref[...]
        u = None
        for k0 in range(0, K2, 256):
            kw = min(256, K2 - k0)
            y = jnp.dot(x_ref[:, k0:k0 + kw], wr_ref[k0:k0 + kw, :],
                        preferred_element_type=jnp.float32)
            u = y if u is None else u + y
        o_ref[...] = t + u

    return pl.pallas_call(
        body,
        grid=(_cdiv(M, bm),),
        in_specs=[
            pl.BlockSpec((bm, K1), lambda i: (i, 0)),
            pl.BlockSpec((K1, N), lambda i: (0, 0)),
            pl.BlockSpec((1, N), lambda i: (0, 0)),
            pl.BlockSpec((bm, K2), lambda i: (i, 0)),
            pl.BlockSpec((K2, N), lambda i: (0, 0)),
        ],
        out_specs=pl.BlockSpec((bm, N), lambda i: (i, 0)),
        out_shape=jax.ShapeDtypeStruct((M, N), jnp.float32),
    )(mean, WlT, bl.reshape(1, N), x, WrT)


_SC_RELS = {"created_by"}
_GRP = {"Asset": 16, "Creator": 64, "Licensee": 128}

_E = 50000
_EPAD = 51200
_ET = _EPAD // 16
_B = 128
_NB = _ET // _B


def _seg_sum_sc(table, src, dst, n_dst, G):
    ns, W = table.shape
    NG = W // G
    NGC = NG // 2
    NdT = n_dst + 16
    rows_z = NdT // 16
    rows_w = n_dst // 16
    ZR = 128

    srcP = jnp.concatenate([src, jnp.zeros((_EPAD - _E,), jnp.int32)])
    dstP = jnp.concatenate([dst, jnp.full((_EPAD - _E,), n_dst, jnp.int32)])
    tview = table.reshape(ns * NG, G)

    mesh = plsc.VectorSubcoreMesh(core_axis_name="c", subcore_axis_name="s",
                                  num_cores=2, num_subcores=16)

    def body(tab_ref, src_ref, dst_ref, out_ref, srcb, dstb, idxb, g0, g1,
             zbuf, acc, sem0, sem1):
        c = lax.axis_index("c")
        s = lax.axis_index("s")
        base = s * _ET

        for b in range(_NB):
            pltpu.sync_copy(src_ref.at[pl.ds(base + b * _B, _B)], srcb.at[b])
            pltpu.sync_copy(dst_ref.at[pl.ds(base + b * _B, _B)], dstb.at[b])

        @pl.loop(0, _NB * (_B // 16))
        def _premul(i):
            b = i // (_B // 16)
            k = (i % (_B // 16)) * 16
            srcb[b, pl.ds(k, 16)] = srcb[b, pl.ds(k, 16)] * NG

        @pl.loop(0, ZR * (G // 16))
        def _zfill(i):
            r = i // (G // 16)
            col = (i % (G // 16)) * 16
            zbuf[r, pl.ds(col, 16)] = jnp.zeros((16,), jnp.float32)

        @pl.loop(0, NGC)
        def _group(gi):
            g = gi * 2 + c

            @pl.loop(0, rows_z // ZR)
            def _zero(z):
                pltpu.sync_copy(zbuf, acc.at[pl.ds(s * rows_z + z * ZR, ZR)])

            if rows_z % ZR:
                pltpu.sync_copy(
                    zbuf.at[pl.ds(0, rows_z % ZR)],
                    acc.at[pl.ds(s * rows_z + (rows_z // ZR) * ZR,
                                 rows_z % ZR)])
            plsc.subcore_barrier()

            @pl.loop(0, _NB * (_B // 16))
            def _addg(i):
                b = i // (_B // 16)
                k = (i % (_B // 16)) * 16
                idxb[b, pl.ds(k, 16)] = srcb[b, pl.ds(k, 16)] + g

            cps = [None] * _NB
            cps[0] = pltpu.async_copy(tab_ref.at[idxb.at[0]], g0, sem0)
            for b in range(_NB):
                buf = g0 if b % 2 == 0 else g1
                if b + 1 < _NB:
                    nbuf = g1 if b % 2 == 0 else g0
                    nsem = sem1 if b % 2 == 0 else sem0
                    cps[b + 1] = pltpu.async_copy(
                        tab_ref.at[idxb.at[b + 1]], nbuf, nsem)
                cps[b].wait()
                pltpu.sync_copy(buf, acc.at[dstb.at[b]], add=True)
            plsc.subcore_barrier()

            pltpu.sync_copy(
                acc.at[pl.ds(s * rows_w, rows_w)],
                out_ref.at[g, pl.ds(s * rows_w, rows_w)])
            plsc.subcore_barrier()

    f = pl.kernel(
        body,
        out_type=jax.ShapeDtypeStruct((NG, n_dst, G), jnp.float32),
        mesh=mesh,
        scratch_types=[
            pltpu.VMEM((_NB, _B), jnp.int32),
            pltpu.VMEM((_NB, _B), jnp.int32),
            pltpu.VMEM((_NB, _B), jnp.int32),
            pltpu.VMEM((_B, G), jnp.float32),
            pltpu.VMEM((_B, G), jnp.float32),
            pltpu.VMEM((ZR, G), jnp.float32),
            pltpu.VMEM_SHARED((NdT, G), jnp.float32),
            pltpu.SemaphoreType.DMA,
            pltpu.SemaphoreType.DMA,
        ],
        compiler_params=pltpu.CompilerParams(use_tc_tiling_on_sc=False),
    )
    return f(tview, srcP, dstP)


def _mm_seg(s3, counts, Wt, bias=None, acc=None, block_m=512):
    NG, M, G = s3.shape
    N = Wt.shape[1]
    bm = min(block_m, M)

    has_acc = acc is not None
    has_bias = bias is not None

    def body(*refs):
        i = 0
        x_ref = refs[i]; i += 1
        w_ref = refs[i]; i += 1
        c_ref = refs[i]; i += 1
        b_ref = None
        a_ref = None
        if has_bias:
            b_ref = refs[i]; i += 1
        if has_acc:
            a_ref = refs[i]; i += 1
        o_ref = refs[i]
        g = pl.program_id(1)
        cm = jnp.maximum(c_ref[...], 1.0)
        y = jnp.dot(x_ref[0] / cm, w_ref[...],
                    preferred_element_type=jnp.float32)

        @pl.when(g == 0)
        def _():
            z = y
            if has_bias:
                z = z + b_ref[...]
            if has_acc:
                z = z + a_ref[...]
            o_ref[...] = z

        @pl.when(g > 0)
        def _():
            o_ref[...] = o_ref[...] + y

    in_specs = [
        pl.BlockSpec((1, bm, G), lambda i, g: (g, i, 0)),
        pl.BlockSpec((G, N), lambda i, g: (g, 0)),
        pl.BlockSpec((bm, 1), lambda i, g: (i, 0)),
    ]
    ops = [s3, Wt, counts.reshape(M, 1)]
    if has_bias:
        in_specs.append(pl.BlockSpec((1, N), lambda i, g: (0, 0)))
        ops.append(bias.reshape(1, N))
    if has_acc:
        in_specs.append(pl.BlockSpec((bm, N), lambda i, g: (i, 0)))
        ops.append(acc)

    return pl.pallas_call(
        body,
        grid=(_cdiv(M, bm), NG),
        in_specs=in_specs,
        out_specs=pl.BlockSpec((bm, N), lambda i, g: (i, 0)),
        out_shape=jax.ShapeDtypeStruct((M, N), jnp.float32),
    )(*ops)


def _seg_mean(x_src, ei, n_dst):
    m = jnp.take(x_src, ei[0], axis=0)
    s = jax.ops.segment_sum(m, ei[1], num_segments=n_dst)
    c = jax.ops.segment_sum(jnp.ones((ei.shape[1],), x_src.dtype), ei[1],
                            num_segments=n_dst)
    return s / jnp.maximum(c, 1.0)[:, None]


def _heads(h2_A, h2_C, params):
    OUT = h2_A.shape[1]
    q = h2_A[0:1, :]
    W1, b1 = params["inf1"]
    W2, b2 = params["inf2"]
    A1, ab1 = params["att1"]
    A2, ab2 = params["att2"]

    def inf_body(q_ref, w1_ref, b1_ref, w2_ref, b2_ref, o_ref):
        t = jnp.dot(q_ref[...], w1_ref[...], preferred_element_type=jnp.float32)
        t = jnp.maximum(t + b1_ref[...], 0.0)
        o_ref[...] = jnp.dot(t, w2_ref[...],
                             preferred_element_type=jnp.float32) + b2_ref[...]

    inf = pl.pallas_call(
        inf_body,
        out_shape=jax.ShapeDtypeStruct((1, 1), jnp.float32),
    )(q, W1.T, b1.reshape(1, OUT), W2.T, b2.reshape(1, 1))

    NC = h2_C.shape[0]
    bm = 1024
    pair = jnp.concatenate(
        [jnp.broadcast_to(q, (NC, OUT)), h2_C], axis=-1)

    def att_body(p_ref, a1_ref, ab1_ref, a2_ref, ab2_ref, o_ref):
        t = jnp.dot(p_ref[...], a1_ref[...], preferred_element_type=jnp.float32)
        t = jnp.maximum(t + ab1_ref[...], 0.0)
        o_ref[...] = jnp.dot(t, a2_ref[...],
                             preferred_element_type=jnp.float32) + ab2_ref[...]

    att = pl.pallas_call(
        att_body,
        grid=(_cdiv(NC, bm),),
        in_specs=[
            pl.BlockSpec((bm, 2 * OUT), lambda i: (i, 0)),
            pl.BlockSpec((2 * OUT, OUT), lambda i: (0, 0)),
            pl.BlockSpec((1, OUT), lambda i: (0, 0)),
            pl.BlockSpec((OUT, 1), lambda i: (0, 0)),
            pl.BlockSpec((1, 1), lambda i: (0, 0)),
        ],
        out_specs=pl.BlockSpec((bm, 1), lambda i: (i, 0)),
        out_shape=jax.ShapeDtypeStruct((NC, 1), jnp.float32),
    )(pair, A1.T, ab1.reshape(1, OUT), A2.T, ab2.reshape(1, 1))

    return inf.reshape(()), att[:, 0]


def kernel(x_Asset, x_Creator, x_Licensee, params, ei_created_by,
           ei_licensed_to, ei_similar_to, ei_flagged_with, ei_rev_created_by,
           ei_rev_licensed_to):
    eis = {
        "created_by": ei_created_by,
        "licensed_to": ei_licensed_to,
        "similar_to": ei_similar_to,
        "flagged_with": ei_flagged_with,
        "rev_created_by": ei_rev_created_by,
        "rev_licensed_to": ei_rev_licensed_to,
    }
    nn = {"Asset": x_Asset.shape[0], "Creator": x_Creator.shape[0],
          "Licensee": x_Licensee.shape[0]}
    p = params

    Wa, ba = p["asset_proj"]
    Wc, bc = p["creator_proj"]
    Wl_, bl_ = p["licensee_proj"]
    h = {
        "Asset": _mm(x_Asset, Wa.T, bias=ba),
        "Creator": _mm(x_Creator, Wc.T, bias=bc),
        "Licensee": _mm(x_Licensee, Wl_.T, bias=bl_),
    }

    for lp in (p["conv1"], p["conv2"]):
        tot = {}
        for name, st, dt in _REL:
            if name in _SC_RELS:
                s3 = _seg_sum_sc(h[st], eis[name][0], eis[name][1], nn[dt],
                                 _GRP[dt])
                cnt = jax.ops.segment_sum(
                    jnp.ones((_E,), jnp.float32), eis[name][1],
                    num_segments=nn[dt])
                s = jnp.concatenate([s3[g] for g in range(s3.shape[0])],
                                    axis=1)
                mean = s / jnp.maximum(cnt, 1.0)[:, None]
            else:
                mean = _seg_mean(h[st], eis[name], nn[dt])
            o = _sage_mm(mean, lp[name]["Wl"].T, lp[name]["bl"], h[dt],
                         lp[name]["Wr"].T)
            tot[dt] = o if dt not in tot else tot[dt] + o
        h = {dt: jax.nn.relu(v) for dt, v in tot.items()}

    inf, att = _heads(h["Asset"], h["Creator"], params)
    return (inf, att, h["Asset"], h["Creator"], h["Licensee"])
